# baseline (speedup 1.0000x reference)
import functools

import jax
import jax.numpy as jnp
from jax import lax
from jax.experimental import pallas as pl
from jax.experimental.pallas import tpu as pltpu
from jax.experimental.pallas import tpu_sc as plsc

K_CODES = 8192
D_DIM = 256
RB = 256
KC = 512
N_ROW_BLOCKS = K_CODES // RB
N_K_CHUNKS = K_CODES // KC

SC_CORES = 2
SC_SUBCORES = 16
SC_WORKERS = SC_CORES * SC_SUBCORES
ROWS_PER_WORKER = K_CODES // SC_WORKERS
IDX_CHUNK = 128
CHUNKS_PER_WORKER = ROWS_PER_WORKER // IDX_CHUNK


def _dist_argmin_body(z_ref, cbt_ref, idx_ref, cnorm_ref):
    i = pl.program_id(0)
    zb = z_ref[...]
    znorm = jnp.sum(zb * zb, axis=1)

    def chunk(j, carry):
        bestv, besti = carry
        c = cbt_ref[:, pl.ds(j * KC, KC)]

        @pl.when(i == 0)
        def _():
            cnorm_ref[:, pl.ds(j * KC, KC)] = jnp.sum(c * c, axis=0)[None, :]

        cn = cnorm_ref[:, pl.ds(j * KC, KC)]
        mm = lax.dot_general(zb, c, (((1,), (0,)), ((), ())),
                             preferred_element_type=jnp.float32)
        d = (znorm[:, None] + cn) - 2.0 * mm
        m = jnp.min(d, axis=1)
        iota = lax.broadcasted_iota(jnp.int32, (RB, KC), 1) + j * KC
        ii = jnp.min(jnp.where(d == m[:, None], iota, K_CODES), axis=1)
        upd = m < bestv
        return jnp.where(upd, m, bestv), jnp.where(upd, ii, besti)

    init = (jnp.full((RB,), jnp.inf, jnp.float32), jnp.zeros((RB,), jnp.int32))
    _, besti = lax.fori_loop(0, N_K_CHUNKS, chunk, init)
    idx_ref[...] = besti.reshape(1, 1, RB)


def _dist_argmin(z_flat, cbt):
    out = pl.pallas_call(
        _dist_argmin_body,
        grid=(N_ROW_BLOCKS,),
        in_specs=[
            pl.BlockSpec((RB, D_DIM), lambda i: (i, 0)),
            pl.BlockSpec((D_DIM, K_CODES), lambda i: (0, 0)),
        ],
        out_specs=pl.BlockSpec((1, 1, RB), lambda i: (i, 0, 0)),
        out_shape=jax.ShapeDtypeStruct((N_ROW_BLOCKS, 1, RB), jnp.int32),
        scratch_shapes=[pltpu.VMEM((1, K_CODES), jnp.float32)],
        compiler_params=pltpu.CompilerParams(
            dimension_semantics=("arbitrary",)),
    )(z_flat, cbt)
    return out.reshape(-1)


def _sc_gather(codebook, idx2d):
    mesh = plsc.VectorSubcoreMesh(
        core_axis_name="c", subcore_axis_name="s",
        num_cores=SC_CORES, num_subcores=SC_SUBCORES)

    @functools.partial(
        pl.kernel,
        out_type=jax.ShapeDtypeStruct((K_CODES, D_DIM), jnp.float32),
        mesh=mesh,
        scratch_types=[
            pltpu.VMEM((CHUNKS_PER_WORKER, IDX_CHUNK), jnp.int32),
            pltpu.VMEM((ROWS_PER_WORKER, D_DIM), jnp.float32),
            pltpu.SemaphoreType.DMA,
        ],
    )
    def gather(table_hbm, idx_hbm, out_hbm, idx_v, rows_v, sem):
        wid = lax.axis_index("s") * SC_CORES + lax.axis_index("c")
        base = wid * ROWS_PER_WORKER
        pltpu.sync_copy(idx_hbm.at[pl.ds(wid * CHUNKS_PER_WORKER,
                                         CHUNKS_PER_WORKER)], idx_v)
        copies = [
            pltpu.async_copy(table_hbm.at[idx_v.at[c]],
                             rows_v.at[pl.ds(c * IDX_CHUNK, IDX_CHUNK)], sem)
            for c in range(CHUNKS_PER_WORKER)
        ]
        for cp in copies:
            cp.wait()
        pltpu.sync_copy(rows_v, out_hbm.at[pl.ds(base, ROWS_PER_WORKER)])

    return gather(codebook, idx2d)


def kernel(z_e, codebook):
    z = jnp.transpose(z_e, (0, 2, 3, 1))
    z_flat = z.reshape(-1, D_DIM)
    cbt = codebook.T

    indices = _dist_argmin(z_flat, cbt)
    g = _sc_gather(codebook, indices.reshape(-1, IDX_CHUNK))

    z_q_flat = z_flat + lax.stop_gradient(g - z_flat)
    z_q = jnp.transpose(z_q_flat.reshape(z.shape), (0, 3, 1, 2))
    idx_out = indices.reshape(z.shape[:-1])
    return (z_e, z_q, idx_out)


# speedup vs baseline: 2.1660x; 2.1660x over previous
import functools

import jax
import jax.numpy as jnp
from jax import lax
from jax.experimental import pallas as pl
from jax.experimental.pallas import tpu as pltpu
from jax.experimental.pallas import tpu_sc as plsc

K_CODES = 8192
D_DIM = 256
RB = 256
KC = 512
N_ROW_BLOCKS = K_CODES // RB
N_K_CHUNKS = K_CODES // KC

SC_CORES = 2
SC_SUBCORES = 16
SC_WORKERS = SC_CORES * SC_SUBCORES
ROWS_PER_WORKER = K_CODES // SC_WORKERS
IDX_CHUNK = 128
CHUNKS_PER_WORKER = ROWS_PER_WORKER // IDX_CHUNK


def _dist_argmin_body(z_ref, cbt_ref, idx_ref, cnorm_ref):
    i = pl.program_id(0)
    zb = z_ref[...]
    znorm = jnp.sum(zb * zb, axis=1, keepdims=True)

    @pl.when(i == 0)
    def _():
        for j in range(N_K_CHUNKS):
            c = cbt_ref[:, pl.ds(j * KC, KC)]
            cnorm_ref[:, pl.ds(j * KC, KC)] = jnp.sum(c * c, axis=0)[None, :]

    iota = lax.broadcasted_iota(jnp.int32, (RB, KC), 1).astype(jnp.float32)
    bestv = jnp.full((RB, 1), jnp.inf, jnp.float32)
    besti = jnp.zeros((RB, 1), jnp.float32)
    for j in range(N_K_CHUNKS):
        c = cbt_ref[:, pl.ds(j * KC, KC)]
        cn = cnorm_ref[:, pl.ds(j * KC, KC)]
        mm = lax.dot_general(zb, c, (((1,), (0,)), ((), ())),
                             preferred_element_type=jnp.float32)
        d = (znorm + cn) - 2.0 * mm
        m = jnp.min(d, axis=1, keepdims=True)
        ii = jnp.min(jnp.where(d == m, iota, jnp.float32(KC)),
                     axis=1, keepdims=True)
        upd = m < bestv
        bestv = jnp.where(upd, m, bestv)
        besti = jnp.where(upd, ii + jnp.float32(j * KC), besti)
    idx_ref[...] = besti.astype(jnp.int32).reshape(1, 1, RB)


def _dist_argmin(z_flat, cbt):
    out = pl.pallas_call(
        _dist_argmin_body,
        grid=(N_ROW_BLOCKS,),
        in_specs=[
            pl.BlockSpec((RB, D_DIM), lambda i: (i, 0)),
            pl.BlockSpec((D_DIM, K_CODES), lambda i: (0, 0)),
        ],
        out_specs=pl.BlockSpec((1, 1, RB), lambda i: (i, 0, 0)),
        out_shape=jax.ShapeDtypeStruct((N_ROW_BLOCKS, 1, RB), jnp.int32),
        scratch_shapes=[pltpu.VMEM((1, K_CODES), jnp.float32)],
        compiler_params=pltpu.CompilerParams(
            dimension_semantics=("arbitrary",)),
    )(z_flat, cbt)
    return out.reshape(-1)


def _sc_gather(codebook, idx2d):
    mesh = plsc.VectorSubcoreMesh(
        core_axis_name="c", subcore_axis_name="s",
        num_cores=SC_CORES, num_subcores=SC_SUBCORES)

    @functools.partial(
        pl.kernel,
        out_type=jax.ShapeDtypeStruct((K_CODES, D_DIM), jnp.float32),
        mesh=mesh,
        scratch_types=[
            pltpu.VMEM((CHUNKS_PER_WORKER, IDX_CHUNK), jnp.int32),
            pltpu.VMEM((ROWS_PER_WORKER, D_DIM), jnp.float32),
            pltpu.SemaphoreType.DMA,
        ],
    )
    def gather(table_hbm, idx_hbm, out_hbm, idx_v, rows_v, sem):
        wid = lax.axis_index("s") * SC_CORES + lax.axis_index("c")
        base = wid * ROWS_PER_WORKER
        pltpu.sync_copy(idx_hbm.at[pl.ds(wid * CHUNKS_PER_WORKER,
                                         CHUNKS_PER_WORKER)], idx_v)
        copies = [
            pltpu.async_copy(table_hbm.at[idx_v.at[c]],
                             rows_v.at[pl.ds(c * IDX_CHUNK, IDX_CHUNK)], sem)
            for c in range(CHUNKS_PER_WORKER)
        ]
        for cp in copies:
            cp.wait()
        pltpu.sync_copy(rows_v, out_hbm.at[pl.ds(base, ROWS_PER_WORKER)])

    return gather(codebook, idx2d)


def kernel(z_e, codebook):
    z = jnp.transpose(z_e, (0, 2, 3, 1))
    z_flat = z.reshape(-1, D_DIM)
    cbt = codebook.T

    indices = _dist_argmin(z_flat, cbt)
    g = _sc_gather(codebook, indices.reshape(-1, IDX_CHUNK))

    z_q_flat = z_flat + lax.stop_gradient(g - z_flat)
    z_q = jnp.transpose(z_q_flat.reshape(z.shape), (0, 3, 1, 2))
    idx_out = indices.reshape(z.shape[:-1])
    return (z_e, z_q, idx_out)
